# Initial kernel scaffold; baseline (speedup 1.0000x reference)
import functools

import jax
import jax.numpy as jnp
from jax import lax
from jax.experimental import pallas as pl
from jax.experimental.pallas import tpu as pltpu
from jax.experimental.pallas import tpu_sc as plsc

BATCH = 16384
EMB = 16
NUM_NUM = 16
UNITS = 128
AUG = 32
CHUNK = 128

_NC = 2
_NS = 16
_NW = _NC * _NS
_BPW = BATCH // _NW
_NCHUNK = _BPW // CHUNK


def _sc_gather_body(t0, t1, t2, idx, g0, g1, g2, idx_v, r0, r1, r2, sem):
    wid = lax.axis_index("s") * _NC + lax.axis_index("c")
    base = wid * _BPW
    pltpu.sync_copy(idx.at[:, pl.ds(wid * _NCHUNK, _NCHUNK)], idx_v)
    copies = []
    for c in range(_NCHUNK):
        copies.append(pltpu.async_copy(
            t0.at[idx_v.at[0, c]], r0.at[pl.ds(c * CHUNK, CHUNK)], sem))
        copies.append(pltpu.async_copy(
            t1.at[idx_v.at[1, c]], r1.at[pl.ds(c * CHUNK, CHUNK)], sem))
        copies.append(pltpu.async_copy(
            t2.at[idx_v.at[2, c]], r2.at[pl.ds(c * CHUNK, CHUNK)], sem))
    for cp in copies:
        cp.wait()
    pltpu.sync_copy(r0, g0.at[pl.ds(base, _BPW)])
    pltpu.sync_copy(r1, g1.at[pl.ds(base, _BPW)])
    pltpu.sync_copy(r2, g2.at[pl.ds(base, _BPW)])


def _sc_gather(t0, t1, t2, idx):
    mesh = plsc.VectorSubcoreMesh(core_axis_name="c", subcore_axis_name="s")
    f = functools.partial(
        pl.kernel,
        mesh=mesh,
        out_type=[jax.ShapeDtypeStruct((BATCH, AUG), jnp.float32)] * 3,
        scratch_types=[
            pltpu.VMEM((3, _NCHUNK, CHUNK), jnp.int32),
            pltpu.VMEM((_BPW, AUG), jnp.float32),
            pltpu.VMEM((_BPW, AUG), jnp.float32),
            pltpu.VMEM((_BPW, AUG), jnp.float32),
            pltpu.SemaphoreType.DMA,
        ],
    )(_sc_gather_body)
    return f(t0, t1, t2, idx)


def _mlp_body(inp_ref, g0_ref, g1_ref, g2_ref, a0, a1, a2, a3, c1, w2, c2,
              w3, c3, w4, c4, w5, c5, out_ref):
    dot = functools.partial(jnp.dot, preferred_element_type=jnp.float32)
    num = inp_ref[:, 3:3 + NUM_NUM]
    h = (dot(g0_ref[...], a0[...]) + dot(g1_ref[...], a1[...])
         + dot(g2_ref[...], a2[...]) + dot(num, a3[...]) + c1[...])
    h = jnp.maximum(h, 0.0)
    h = jnp.maximum(dot(h, w2[...]) + c2[...], 0.0)
    h = jnp.maximum(dot(h, w3[...]) + c3[...], 0.0)
    h = jnp.maximum(dot(h, w4[...]) + c4[...], 0.0)
    logits = dot(h, w5[...]) + c5[...]
    m = jnp.max(logits, axis=-1, keepdims=True)
    e = jnp.exp(logits - m)
    out_ref[...] = e / jnp.sum(e, axis=-1, keepdims=True)


def _full(shape):
    return pl.BlockSpec(shape, lambda i: (0, 0))


def _mlp(inputs, g0, g1, g2, a0, a1, a2, a3, c1, w2, c2, w3, c3, w4, c4,
         w5, c5, block_b):
    nlab = w5.shape[1]
    grid = (BATCH // block_b,)
    in_specs = [
        pl.BlockSpec((block_b, inputs.shape[1]), lambda i: (i, 0)),
        pl.BlockSpec((block_b, AUG), lambda i: (i, 0)),
        pl.BlockSpec((block_b, AUG), lambda i: (i, 0)),
        pl.BlockSpec((block_b, AUG), lambda i: (i, 0)),
        _full(a0.shape), _full(a1.shape), _full(a2.shape), _full(a3.shape),
        _full(c1.shape), _full(w2.shape), _full(c2.shape), _full(w3.shape),
        _full(c3.shape), _full(w4.shape), _full(c4.shape), _full(w5.shape),
        _full(c5.shape),
    ]
    return pl.pallas_call(
        _mlp_body,
        grid=grid,
        in_specs=in_specs,
        out_specs=pl.BlockSpec((block_b, nlab), lambda i: (i, 0)),
        out_shape=jax.ShapeDtypeStruct((BATCH, nlab), jnp.float32),
        compiler_params=pltpu.CompilerParams(
            dimension_semantics=("arbitrary",)),
    )(inputs, g0, g1, g2, a0, a1, a2, a3, c1, w2, c2, w3, c3, w4, c4, w5, c5)


def kernel(inputs, speed_emb, speed_bias, oneway_emb, oneway_bias, lane_emb,
           lane_bias, W1, b1, W2, b2, W3, b3, W4, b4, W5, b5):
    f32 = jnp.float32
    def aug(emb, bias):
        n = emb.shape[0]
        return jnp.concatenate(
            [emb, bias.reshape(n, 1),
             jnp.zeros((n, AUG - EMB - 1), f32)], axis=1)

    t0 = aug(speed_emb, speed_bias)
    t1 = aug(oneway_emb, oneway_bias)
    t2 = aug(lane_emb, lane_bias)

    idx = inputs[:, 0:3].astype(jnp.int32).T.reshape(3, BATCH // CHUNK, CHUNK)

    g0, g1, g2 = _sc_gather(t0, t1, t2, idx)

    w1s = jnp.sum(W1[:3 * EMB], axis=0, keepdims=True)
    zpad = jnp.zeros((AUG - EMB - 1, UNITS), f32)
    a0 = jnp.concatenate([W1[0:EMB], w1s, zpad], axis=0)
    a1 = jnp.concatenate([W1[EMB:2 * EMB], w1s, zpad], axis=0)
    a2 = jnp.concatenate([W1[2 * EMB:3 * EMB], w1s, zpad], axis=0)
    a3 = W1[3 * EMB:]

    return _mlp(inputs, g0, g1, g2, a0, a1, a2, a3,
                b1.reshape(1, -1), W2, b2.reshape(1, -1),
                W3, b3.reshape(1, -1), W4, b4.reshape(1, -1),
                W5, b5.reshape(1, -1), block_b=512)


# speedup vs baseline: 1.8488x; 1.8488x over previous
import functools

import jax
import jax.numpy as jnp
from jax import lax
from jax.experimental import pallas as pl
from jax.experimental.pallas import tpu as pltpu
from jax.experimental.pallas import tpu_sc as plsc

BATCH = 16384
EMB = 16
NUM_NUM = 16
UNITS = 128
AUG = 32
CHUNK = 128

_NC = 2
_NS = 16
_NW = _NC * _NS
_BPW = BATCH // _NW
_NCHUNK = _BPW // CHUNK


def _sc_gather_body(t0, t1, t2, idx, g0, g1, g2, idx_v, r0, r1, r2, sem):
    wid = lax.axis_index("s") * _NC + lax.axis_index("c")
    base = wid * _BPW
    pltpu.sync_copy(idx.at[:, pl.ds(wid * _NCHUNK, _NCHUNK)], idx_v)
    copies = []
    for c in range(_NCHUNK):
        copies.append(pltpu.async_copy(
            t0.at[idx_v.at[0, c]], r0.at[pl.ds(c * CHUNK, CHUNK)], sem))
        copies.append(pltpu.async_copy(
            t1.at[idx_v.at[1, c]], r1.at[pl.ds(c * CHUNK, CHUNK)], sem))
        copies.append(pltpu.async_copy(
            t2.at[idx_v.at[2, c]], r2.at[pl.ds(c * CHUNK, CHUNK)], sem))
    for cp in copies:
        cp.wait()
    pltpu.sync_copy(r0, g0.at[pl.ds(base, _BPW)])
    pltpu.sync_copy(r1, g1.at[pl.ds(base, _BPW)])
    pltpu.sync_copy(r2, g2.at[pl.ds(base, _BPW)])


def _sc_gather(t0, t1, t2, idx):
    mesh = plsc.VectorSubcoreMesh(core_axis_name="c", subcore_axis_name="s")
    f = functools.partial(
        pl.kernel,
        mesh=mesh,
        out_type=[jax.ShapeDtypeStruct((BATCH, AUG), jnp.float32)] * 3,
        scratch_types=[
            pltpu.VMEM((3, _NCHUNK, CHUNK), jnp.int32),
            pltpu.VMEM((_BPW, AUG), jnp.float32),
            pltpu.VMEM((_BPW, AUG), jnp.float32),
            pltpu.VMEM((_BPW, AUG), jnp.float32),
            pltpu.SemaphoreType.DMA,
        ],
        compiler_params=pltpu.CompilerParams(use_tc_tiling_on_sc=False),
    )(_sc_gather_body)
    return f(t0, t1, t2, idx)


def _mlp_body(inp_ref, g0_ref, g1_ref, g2_ref, a0, a1, a2, a3, c1, w2, c2,
              w3, c3, w4, c4, w5, c5, out_ref):
    dot = functools.partial(jnp.dot, preferred_element_type=jnp.float32)
    num = inp_ref[:, 3:3 + NUM_NUM]
    h = (dot(g0_ref[...], a0[...]) + dot(g1_ref[...], a1[...])
         + dot(g2_ref[...], a2[...]) + dot(num, a3[...]) + c1[...])
    h = jnp.maximum(h, 0.0)
    h = jnp.maximum(dot(h, w2[...]) + c2[...], 0.0)
    h = jnp.maximum(dot(h, w3[...]) + c3[...], 0.0)
    h = jnp.maximum(dot(h, w4[...]) + c4[...], 0.0)
    logits = dot(h, w5[...]) + c5[...]
    m = jnp.max(logits, axis=-1, keepdims=True)
    e = jnp.exp(logits - m)
    out_ref[...] = e / jnp.sum(e, axis=-1, keepdims=True)


def _full(shape):
    return pl.BlockSpec(shape, lambda i: (0, 0))


def _mlp(inputs, g0, g1, g2, a0, a1, a2, a3, c1, w2, c2, w3, c3, w4, c4,
         w5, c5, block_b):
    nlab = w5.shape[1]
    grid = (BATCH // block_b,)
    in_specs = [
        pl.BlockSpec((block_b, inputs.shape[1]), lambda i: (i, 0)),
        pl.BlockSpec((block_b, AUG), lambda i: (i, 0)),
        pl.BlockSpec((block_b, AUG), lambda i: (i, 0)),
        pl.BlockSpec((block_b, AUG), lambda i: (i, 0)),
        _full(a0.shape), _full(a1.shape), _full(a2.shape), _full(a3.shape),
        _full(c1.shape), _full(w2.shape), _full(c2.shape), _full(w3.shape),
        _full(c3.shape), _full(w4.shape), _full(c4.shape), _full(w5.shape),
        _full(c5.shape),
    ]
    return pl.pallas_call(
        _mlp_body,
        grid=grid,
        in_specs=in_specs,
        out_specs=pl.BlockSpec((block_b, nlab), lambda i: (i, 0)),
        out_shape=jax.ShapeDtypeStruct((BATCH, nlab), jnp.float32),
        compiler_params=pltpu.CompilerParams(
            dimension_semantics=("arbitrary",)),
    )(inputs, g0, g1, g2, a0, a1, a2, a3, c1, w2, c2, w3, c3, w4, c4, w5, c5)


def kernel(inputs, speed_emb, speed_bias, oneway_emb, oneway_bias, lane_emb,
           lane_bias, W1, b1, W2, b2, W3, b3, W4, b4, W5, b5):
    f32 = jnp.float32
    def aug(emb, bias):
        n = emb.shape[0]
        return jnp.concatenate(
            [emb, bias.reshape(n, 1),
             jnp.zeros((n, AUG - EMB - 1), f32)], axis=1)

    t0 = aug(speed_emb, speed_bias)
    t1 = aug(oneway_emb, oneway_bias)
    t2 = aug(lane_emb, lane_bias)

    idx = inputs[:, 0:3].astype(jnp.int32).T.reshape(3, BATCH // CHUNK, CHUNK)

    g0, g1, g2 = _sc_gather(t0, t1, t2, idx)

    w1s = jnp.sum(W1[:3 * EMB], axis=0, keepdims=True)
    zpad = jnp.zeros((AUG - EMB - 1, UNITS), f32)
    a0 = jnp.concatenate([W1[0:EMB], w1s, zpad], axis=0)
    a1 = jnp.concatenate([W1[EMB:2 * EMB], w1s, zpad], axis=0)
    a2 = jnp.concatenate([W1[2 * EMB:3 * EMB], w1s, zpad], axis=0)
    a3 = W1[3 * EMB:]

    return _mlp(inputs, g0, g1, g2, a0, a1, a2, a3,
                b1.reshape(1, -1), W2, b2.reshape(1, -1),
                W3, b3.reshape(1, -1), W4, b4.reshape(1, -1),
                W5, b5.reshape(1, -1), block_b=512)


# speedup vs baseline: 2.7944x; 1.5115x over previous
import functools

import jax
import jax.numpy as jnp
from jax import lax
from jax.experimental import pallas as pl
from jax.experimental.pallas import tpu as pltpu
from jax.experimental.pallas import tpu_sc as plsc

BATCH = 16384
EMB = 16
NUM_NUM = 16
UNITS = 128
GCOLS = 3 * (EMB + 1)
IN_W = 3 + NUM_NUM

_NC = 2
_NS = 16
_NW = _NC * _NS
_BPW = BATCH // _NW
_L = 16


def _splat(v):
    return jnp.full((_L,), v, dtype=jnp.int32)


def _sc_gather_body(inputs_hbm, e0, e1, e2, sb0, sb1, sb2, g_hbm,
                    inp_v, t0v, t1v, t2v, b0v, b1v, b2v, g_v):
    wid = lax.axis_index("s") * _NC + lax.axis_index("c")
    base = wid * _BPW
    pltpu.sync_copy(inputs_hbm.at[pl.ds(base, _BPW)], inp_v)
    pltpu.sync_copy(e0, t0v)
    pltpu.sync_copy(e1, t1v)
    pltpu.sync_copy(e2, t2v)
    pltpu.sync_copy(sb0, b0v)
    pltpu.sync_copy(sb1, b1v)
    pltpu.sync_copy(sb2, b2v)

    tables = ((t0v, b0v, 0), (t1v, b1v, EMB + 1), (t2v, b2v, 2 * (EMB + 1)))

    def chunk(k, carry):
        row = k * _L + lax.iota(jnp.int32, _L)
        for c, (tv, bv, col0) in enumerate(tables):
            idx = plsc.load_gather(inp_v, [row, _splat(c)]).astype(jnp.int32)
            for j in range(EMB):
                vals = plsc.load_gather(tv, [idx, _splat(j)])
                plsc.store_scatter(g_v, [row, _splat(col0 + j)], vals)
            bvals = plsc.load_gather(bv, [idx])
            plsc.store_scatter(g_v, [row, _splat(col0 + EMB)], bvals)
        return carry

    lax.fori_loop(0, _BPW // _L, chunk, 0)
    pltpu.sync_copy(g_v, g_hbm.at[pl.ds(base, _BPW)])


def _sc_gather(inputs, e0, e1, e2, sb0, sb1, sb2):
    mesh = plsc.VectorSubcoreMesh(core_axis_name="c", subcore_axis_name="s")
    f = functools.partial(
        pl.kernel,
        mesh=mesh,
        out_type=jax.ShapeDtypeStruct((BATCH, GCOLS), jnp.float32),
        scratch_types=[
            pltpu.VMEM((_BPW, IN_W), jnp.float32),
            pltpu.VMEM(e0.shape, jnp.float32),
            pltpu.VMEM(e1.shape, jnp.float32),
            pltpu.VMEM(e2.shape, jnp.float32),
            pltpu.VMEM(sb0.shape, jnp.float32),
            pltpu.VMEM(sb1.shape, jnp.float32),
            pltpu.VMEM(sb2.shape, jnp.float32),
            pltpu.VMEM((_BPW, GCOLS), jnp.float32),
        ],
        compiler_params=pltpu.CompilerParams(use_tc_tiling_on_sc=False,
                                             needs_layout_passes=False),
    )(_sc_gather_body)
    return f(inputs, e0, e1, e2, sb0, sb1, sb2)


def _mlp_body(inp_ref, g_ref, a1, w1n, c1, w2, c2, w3, c3, w4, c4, w5, c5,
              out_ref):
    dot = functools.partial(jnp.dot, preferred_element_type=jnp.float32)
    num = inp_ref[:, 3:3 + NUM_NUM]
    h = dot(g_ref[...], a1[...]) + dot(num, w1n[...]) + c1[...]
    h = jnp.maximum(h, 0.0)
    h = jnp.maximum(dot(h, w2[...]) + c2[...], 0.0)
    h = jnp.maximum(dot(h, w3[...]) + c3[...], 0.0)
    h = jnp.maximum(dot(h, w4[...]) + c4[...], 0.0)
    logits = dot(h, w5[...]) + c5[...]
    m = jnp.max(logits, axis=-1, keepdims=True)
    e = jnp.exp(logits - m)
    out_ref[...] = e / jnp.sum(e, axis=-1, keepdims=True)


def _full(shape):
    return pl.BlockSpec(shape, lambda i: (0, 0))


def _mlp(inputs, g, a1, w1n, c1, w2, c2, w3, c3, w4, c4, w5, c5, block_b):
    nlab = w5.shape[1]
    grid = (BATCH // block_b,)
    in_specs = [
        pl.BlockSpec((block_b, inputs.shape[1]), lambda i: (i, 0)),
        pl.BlockSpec((block_b, GCOLS), lambda i: (i, 0)),
        _full(a1.shape), _full(w1n.shape), _full(c1.shape),
        _full(w2.shape), _full(c2.shape), _full(w3.shape), _full(c3.shape),
        _full(w4.shape), _full(c4.shape), _full(w5.shape), _full(c5.shape),
    ]
    return pl.pallas_call(
        _mlp_body,
        grid=grid,
        in_specs=in_specs,
        out_specs=pl.BlockSpec((block_b, nlab), lambda i: (i, 0)),
        out_shape=jax.ShapeDtypeStruct((BATCH, nlab), jnp.float32),
        compiler_params=pltpu.CompilerParams(
            dimension_semantics=("arbitrary",)),
    )(inputs, g, a1, w1n, c1, w2, c2, w3, c3, w4, c4, w5, c5)


def kernel(inputs, speed_emb, speed_bias, oneway_emb, oneway_bias, lane_emb,
           lane_bias, W1, b1, W2, b2, W3, b3, W4, b4, W5, b5):
    g = _sc_gather(inputs, speed_emb, oneway_emb, lane_emb,
                   speed_bias.reshape(-1), oneway_bias.reshape(-1),
                   lane_bias.reshape(-1))

    w1s = jnp.sum(W1[:3 * EMB], axis=0, keepdims=True)
    a1 = jnp.concatenate([W1[0:EMB], w1s, W1[EMB:2 * EMB], w1s,
                          W1[2 * EMB:3 * EMB], w1s], axis=0)
    w1n = W1[3 * EMB:]

    return _mlp(inputs, g, a1, w1n,
                b1.reshape(1, -1), W2, b2.reshape(1, -1),
                W3, b3.reshape(1, -1), W4, b4.reshape(1, -1),
                W5, b5.reshape(1, -1), block_b=512)


# speedup vs baseline: 3.4562x; 1.2368x over previous
import functools

import jax
import jax.numpy as jnp
from jax import lax
from jax.experimental import pallas as pl
from jax.experimental.pallas import tpu as pltpu
from jax.experimental.pallas import tpu_sc as plsc

BATCH = 16384
EMB = 16
NUM_NUM = 16
UNITS = 128
GCOLS = 3 * (EMB + 1)

_NC = 2
_NS = 16
_NW = _NC * _NS
_BPW = BATCH // _NW
_L = 16
_IPR = 4


def _sc_gather_body(idx_hbm, t0, t1, t2, b0, b1, b2, g_hbm,
                    idx_v, t0v, t1v, t2v, b0v, b1v, b2v, g_v):
    wid = lax.axis_index("s") * _NC + lax.axis_index("c")
    pltpu.sync_copy(idx_hbm.at[wid], idx_v)
    pltpu.sync_copy(t0, t0v)
    pltpu.sync_copy(t1, t1v)
    pltpu.sync_copy(t2, t2v)
    pltpu.sync_copy(b0, b0v)
    pltpu.sync_copy(b1, b1v)
    pltpu.sync_copy(b2, b2v)

    tables = ((t0v, b0v, 0), (t1v, b1v, EMB + 1), (t2v, b2v, 2 * (EMB + 1)))
    for kk in range(_IPR):
        for c in range(8):
            k = kk * 8 + c
            for t, (tv, bv, row0) in enumerate(tables):
                iv = idx_v[t * _IPR + kk, pl.ds(c * _L, _L)]
                lin = iv * EMB
                for j in range(EMB):
                    lj = lin + j
                    vals = plsc.load_gather(
                        tv, [lax.shift_right_logical(lj, 7),
                             lax.bitwise_and(lj, 127)])
                    g_v[row0 + j, pl.ds(k * _L, _L)] = vals
                bvals = plsc.load_gather(
                    bv, [lax.shift_right_logical(iv, 7),
                         lax.bitwise_and(iv, 127)])
                g_v[row0 + EMB, pl.ds(k * _L, _L)] = bvals

    pltpu.sync_copy(g_v, g_hbm.at[:, pl.ds(wid * _BPW, _BPW)])


def _sc_gather(idxw, t0, t1, t2, b0, b1, b2):
    mesh = plsc.VectorSubcoreMesh(core_axis_name="c", subcore_axis_name="s")
    f = functools.partial(
        pl.kernel,
        mesh=mesh,
        out_type=jax.ShapeDtypeStruct((GCOLS, BATCH), jnp.float32),
        scratch_types=[
            pltpu.VMEM((3 * _IPR, 128), jnp.int32),
            pltpu.VMEM(t0.shape, jnp.float32),
            pltpu.VMEM(t1.shape, jnp.float32),
            pltpu.VMEM(t2.shape, jnp.float32),
            pltpu.VMEM(b0.shape, jnp.float32),
            pltpu.VMEM(b1.shape, jnp.float32),
            pltpu.VMEM(b2.shape, jnp.float32),
            pltpu.VMEM((GCOLS, _BPW), jnp.float32),
        ],
        compiler_params=pltpu.CompilerParams(needs_layout_passes=False),
    )(_sc_gather_body)
    return f(idxw, t0, t1, t2, b0, b1, b2)


def _mlp_body(inp_ref, g_ref, a1, w1n, c1, w2, c2, w3, c3, w4, c4, w5, c5,
              out_ref):
    dot = functools.partial(jnp.dot, preferred_element_type=jnp.float32)
    num = inp_ref[:, 3:3 + NUM_NUM]
    h = lax.dot_general(g_ref[...], a1[...], (((0,), (0,)), ((), ())),
                        preferred_element_type=jnp.float32)
    h = h + dot(num, w1n[...]) + c1[...]
    h = jnp.maximum(h, 0.0)
    h = jnp.maximum(dot(h, w2[...]) + c2[...], 0.0)
    h = jnp.maximum(dot(h, w3[...]) + c3[...], 0.0)
    h = jnp.maximum(dot(h, w4[...]) + c4[...], 0.0)
    logits = dot(h, w5[...]) + c5[...]
    m = jnp.max(logits, axis=-1, keepdims=True)
    e = jnp.exp(logits - m)
    out_ref[...] = e / jnp.sum(e, axis=-1, keepdims=True)


def _full(shape):
    return pl.BlockSpec(shape, lambda i: (0, 0))


def _mlp(inputs, g, a1, w1n, c1, w2, c2, w3, c3, w4, c4, w5, c5, block_b):
    nlab = w5.shape[1]
    grid = (BATCH // block_b,)
    in_specs = [
        pl.BlockSpec((block_b, inputs.shape[1]), lambda i: (i, 0)),
        pl.BlockSpec((GCOLS, block_b), lambda i: (0, i)),
        _full(a1.shape), _full(w1n.shape), _full(c1.shape),
        _full(w2.shape), _full(c2.shape), _full(w3.shape), _full(c3.shape),
        _full(w4.shape), _full(c4.shape), _full(w5.shape), _full(c5.shape),
    ]
    return pl.pallas_call(
        _mlp_body,
        grid=grid,
        in_specs=in_specs,
        out_specs=pl.BlockSpec((block_b, nlab), lambda i: (i, 0)),
        out_shape=jax.ShapeDtypeStruct((BATCH, nlab), jnp.float32),
        compiler_params=pltpu.CompilerParams(
            dimension_semantics=("arbitrary",)),
    )(inputs, g, a1, w1n, c1, w2, c2, w3, c3, w4, c4, w5, c5)


def kernel(inputs, speed_emb, speed_bias, oneway_emb, oneway_bias, lane_emb,
           lane_bias, W1, b1, W2, b2, W3, b3, W4, b4, W5, b5):
    idx3 = inputs[:, 0:3].astype(jnp.int32)
    idxw = (idx3.T.reshape(3, _NW, _IPR, 128)
            .transpose(1, 0, 2, 3).reshape(_NW, 3 * _IPR, 128))

    t0 = speed_emb.reshape(-1, 128)
    t1 = oneway_emb.reshape(-1, 128)
    t2 = lane_emb.reshape(-1, 128)

    def padbias(bias):
        v = bias.reshape(-1)
        n = v.shape[0]
        pad = (-n) % 128
        return jnp.pad(v, (0, pad)).reshape(-1, 128)

    b0 = padbias(speed_bias)
    b1v = padbias(oneway_bias)
    b2v = padbias(lane_bias)

    g = _sc_gather(idxw, t0, t1, t2, b0, b1v, b2v)

    w1s = jnp.sum(W1[:3 * EMB], axis=0, keepdims=True)
    a1 = jnp.concatenate([W1[0:EMB], w1s, W1[EMB:2 * EMB], w1s,
                          W1[2 * EMB:3 * EMB], w1s], axis=0)
    w1n = W1[3 * EMB:]

    return _mlp(inputs, g, a1, w1n,
                b1.reshape(1, -1), W2, b2.reshape(1, -1),
                W3, b3.reshape(1, -1), W4, b4.reshape(1, -1),
                W5, b5.reshape(1, -1), block_b=512)


# speedup vs baseline: 4.0066x; 1.1592x over previous
import functools

import jax
import jax.numpy as jnp
from jax import lax
from jax.experimental import pallas as pl
from jax.experimental.pallas import tpu as pltpu
from jax.experimental.pallas import tpu_sc as plsc

BATCH = 16384
EMB = 16
NUM_NUM = 16
UNITS = 128
GCOLS = 3 * (EMB + 1)

_NC = 2
_NS = 16
_NW = _NC * _NS
_BPW = BATCH // _NW
_L = 16
_IPR = 4


def _sc_gather_body(idx_hbm, t0, t1, t2, b0, b1, b2, g_hbm,
                    idx_v, t0v, t1v, t2v, b0v, b1v, b2v, g_v):
    wid = lax.axis_index("s") * _NC + lax.axis_index("c")
    pltpu.sync_copy(idx_hbm.at[wid], idx_v)
    pltpu.sync_copy(t0, t0v)
    pltpu.sync_copy(t1, t1v)
    pltpu.sync_copy(t2, t2v)
    pltpu.sync_copy(b0, b0v)
    pltpu.sync_copy(b1, b1v)
    pltpu.sync_copy(b2, b2v)

    tables = ((t0v, b0v, 0), (t1v, b1v, EMB + 1), (t2v, b2v, 2 * (EMB + 1)))
    for kk in range(_IPR):
        for c in range(8):
            k = kk * 8 + c
            for t, (tv, bv, row0) in enumerate(tables):
                iv = idx_v[t * _IPR + kk, pl.ds(c * _L, _L)]
                lin = iv * EMB
                for j in range(EMB):
                    lj = lin + j
                    vals = plsc.load_gather(
                        tv, [lax.shift_right_logical(lj, 7),
                             lax.bitwise_and(lj, 127)])
                    g_v[row0 + j, pl.ds(k * _L, _L)] = vals
                bvals = plsc.load_gather(
                    bv, [lax.shift_right_logical(iv, 7),
                         lax.bitwise_and(iv, 127)])
                g_v[row0 + EMB, pl.ds(k * _L, _L)] = bvals

    pltpu.sync_copy(g_v, g_hbm.at[:, pl.ds(wid * _BPW, _BPW)])


def _sc_gather(idxw, t0, t1, t2, b0, b1, b2):
    mesh = plsc.VectorSubcoreMesh(core_axis_name="c", subcore_axis_name="s")
    f = functools.partial(
        pl.kernel,
        mesh=mesh,
        out_type=jax.ShapeDtypeStruct((GCOLS, BATCH), jnp.float32),
        scratch_types=[
            pltpu.VMEM((3 * _IPR, 128), jnp.int32),
            pltpu.VMEM(t0.shape, jnp.float32),
            pltpu.VMEM(t1.shape, jnp.float32),
            pltpu.VMEM(t2.shape, jnp.float32),
            pltpu.VMEM(b0.shape, jnp.float32),
            pltpu.VMEM(b1.shape, jnp.float32),
            pltpu.VMEM(b2.shape, jnp.float32),
            pltpu.VMEM((GCOLS, _BPW), jnp.float32),
        ],
        compiler_params=pltpu.CompilerParams(needs_layout_passes=False),
    )(_sc_gather_body)
    return f(idxw, t0, t1, t2, b0, b1, b2)


def _mlp_body(inp_ref, g_ref, a1, w1n, c1, w2, c2, w3, c3, w4, c4, w5, c5,
              out_ref):
    dot = functools.partial(jnp.dot, preferred_element_type=jnp.float32)
    num = inp_ref[:, 3:3 + NUM_NUM]
    h = lax.dot_general(g_ref[...], a1[...], (((0,), (0,)), ((), ())),
                        preferred_element_type=jnp.float32)
    h = h + dot(num, w1n[...]) + c1[...]
    h = jnp.maximum(h, 0.0)
    h = jnp.maximum(dot(h, w2[...]) + c2[...], 0.0)
    h = jnp.maximum(dot(h, w3[...]) + c3[...], 0.0)
    h = jnp.maximum(dot(h, w4[...]) + c4[...], 0.0)
    logits = dot(h, w5[...]) + c5[...]
    m = jnp.max(logits, axis=-1, keepdims=True)
    e = jnp.exp(logits - m)
    out_ref[...] = e / jnp.sum(e, axis=-1, keepdims=True)


def _full(shape):
    return pl.BlockSpec(shape, lambda i: (0, 0))


def _mlp(inputs, g, a1, w1n, c1, w2, c2, w3, c3, w4, c4, w5, c5, block_b):
    nlab = w5.shape[1]
    grid = (BATCH // block_b,)
    in_specs = [
        pl.BlockSpec((block_b, inputs.shape[1]), lambda i: (i, 0)),
        pl.BlockSpec((GCOLS, block_b), lambda i: (0, i)),
        _full(a1.shape), _full(w1n.shape), _full(c1.shape),
        _full(w2.shape), _full(c2.shape), _full(w3.shape), _full(c3.shape),
        _full(w4.shape), _full(c4.shape), _full(w5.shape), _full(c5.shape),
    ]
    return pl.pallas_call(
        _mlp_body,
        grid=grid,
        in_specs=in_specs,
        out_specs=pl.BlockSpec((block_b, nlab), lambda i: (i, 0)),
        out_shape=jax.ShapeDtypeStruct((BATCH, nlab), jnp.float32),
        compiler_params=pltpu.CompilerParams(
            dimension_semantics=("arbitrary",)),
    )(inputs, g, a1, w1n, c1, w2, c2, w3, c3, w4, c4, w5, c5)


def kernel(inputs, speed_emb, speed_bias, oneway_emb, oneway_bias, lane_emb,
           lane_bias, W1, b1, W2, b2, W3, b3, W4, b4, W5, b5):
    idx3 = inputs[:, 0:3].astype(jnp.int32)
    idxw = (idx3.T.reshape(3, _NW, _IPR, 128)
            .transpose(1, 0, 2, 3).reshape(_NW, 3 * _IPR, 128))

    t0 = speed_emb.reshape(-1, 128)
    t1 = oneway_emb.reshape(-1, 128)
    t2 = lane_emb.reshape(-1, 128)

    def padbias(bias):
        v = bias.reshape(-1)
        n = v.shape[0]
        pad = (-n) % 128
        return jnp.pad(v, (0, pad)).reshape(-1, 128)

    b0 = padbias(speed_bias)
    b1v = padbias(oneway_bias)
    b2v = padbias(lane_bias)

    g = _sc_gather(idxw, t0, t1, t2, b0, b1v, b2v)

    w1s = jnp.sum(W1[:3 * EMB], axis=0, keepdims=True)
    a1 = jnp.concatenate([W1[0:EMB], w1s, W1[EMB:2 * EMB], w1s,
                          W1[2 * EMB:3 * EMB], w1s], axis=0)
    w1n = W1[3 * EMB:]

    return _mlp(inputs, g, a1, w1n,
                b1.reshape(1, -1), W2, b2.reshape(1, -1),
                W3, b3.reshape(1, -1), W4, b4.reshape(1, -1),
                W5, b5.reshape(1, -1), block_b=1024)


# speedup vs baseline: 4.3153x; 1.0771x over previous
import functools

import jax
import jax.numpy as jnp
from jax import lax
from jax.experimental import pallas as pl
from jax.experimental.pallas import tpu as pltpu
from jax.experimental.pallas import tpu_sc as plsc

BATCH = 16384
EMB = 16
NUM_NUM = 16
UNITS = 128
GCOLS = 3 * (EMB + 1)

_NC = 2
_NS = 16
_NW = _NC * _NS
_BPW = BATCH // _NW
_L = 16
_IPR = 4


def _sc_gather_body(idx_hbm, t0, t1, t2, b0, b1, b2, g_hbm,
                    idx_v, t0v, t1v, t2v, b0v, b1v, b2v, g_v):
    wid = lax.axis_index("s") * _NC + lax.axis_index("c")
    pltpu.sync_copy(idx_hbm.at[wid], idx_v)
    pltpu.sync_copy(t0, t0v)
    pltpu.sync_copy(t1, t1v)
    pltpu.sync_copy(t2, t2v)
    pltpu.sync_copy(b0, b0v)
    pltpu.sync_copy(b1, b1v)
    pltpu.sync_copy(b2, b2v)

    tables = ((t0v, b0v, 0), (t1v, b1v, EMB + 1), (t2v, b2v, 2 * (EMB + 1)))
    for kk in range(_IPR):
        for c in range(8):
            k = kk * 8 + c
            for t, (tv, bv, row0) in enumerate(tables):
                iv = idx_v[t * _IPR + kk, pl.ds(c * _L, _L)]
                lin = iv * EMB
                for j in range(EMB):
                    lj = lin + j
                    vals = plsc.load_gather(
                        tv, [lax.shift_right_logical(lj, 7),
                             lax.bitwise_and(lj, 127)])
                    g_v[row0 + j, pl.ds(k * _L, _L)] = vals
                bvals = plsc.load_gather(
                    bv, [lax.shift_right_logical(iv, 7),
                         lax.bitwise_and(iv, 127)])
                g_v[row0 + EMB, pl.ds(k * _L, _L)] = bvals

    pltpu.sync_copy(g_v, g_hbm.at[:, pl.ds(wid * _BPW, _BPW)])


def _sc_gather(idxw, t0, t1, t2, b0, b1, b2):
    mesh = plsc.VectorSubcoreMesh(core_axis_name="c", subcore_axis_name="s")
    f = functools.partial(
        pl.kernel,
        mesh=mesh,
        out_type=jax.ShapeDtypeStruct((GCOLS, BATCH), jnp.float32),
        scratch_types=[
            pltpu.VMEM((3 * _IPR, 128), jnp.int32),
            pltpu.VMEM(t0.shape, jnp.float32),
            pltpu.VMEM(t1.shape, jnp.float32),
            pltpu.VMEM(t2.shape, jnp.float32),
            pltpu.VMEM(b0.shape, jnp.float32),
            pltpu.VMEM(b1.shape, jnp.float32),
            pltpu.VMEM(b2.shape, jnp.float32),
            pltpu.VMEM((GCOLS, _BPW), jnp.float32),
        ],
        compiler_params=pltpu.CompilerParams(needs_layout_passes=False),
    )(_sc_gather_body)
    return f(idxw, t0, t1, t2, b0, b1, b2)


def _mlp_body(inp_ref, g_ref, a1, w1n, c1, w2, c2, w3, c3, w4, c4, w5, c5,
              out_ref):
    dot = functools.partial(jnp.dot, preferred_element_type=jnp.float32)
    num = inp_ref[:, 3:3 + NUM_NUM]
    h = lax.dot_general(g_ref[...], a1[...], (((0,), (0,)), ((), ())),
                        preferred_element_type=jnp.float32)
    h = h + dot(num, w1n[...]) + c1[...]
    h = jnp.maximum(h, 0.0)
    h = jnp.maximum(dot(h, w2[...]) + c2[...], 0.0)
    h = jnp.maximum(dot(h, w3[...]) + c3[...], 0.0)
    h = jnp.maximum(dot(h, w4[...]) + c4[...], 0.0)
    logits = dot(h, w5[...]) + c5[...]
    m = jnp.max(logits, axis=-1, keepdims=True)
    e = jnp.exp(logits - m)
    out_ref[...] = e / jnp.sum(e, axis=-1, keepdims=True)


def _full(shape):
    return pl.BlockSpec(shape, lambda i: (0, 0))


def _mlp(inputs, g, a1, w1n, c1, w2, c2, w3, c3, w4, c4, w5, c5, block_b):
    nlab = w5.shape[1]
    grid = (BATCH // block_b,)
    in_specs = [
        pl.BlockSpec((block_b, inputs.shape[1]), lambda i: (i, 0)),
        pl.BlockSpec((GCOLS, block_b), lambda i: (0, i)),
        _full(a1.shape), _full(w1n.shape), _full(c1.shape),
        _full(w2.shape), _full(c2.shape), _full(w3.shape), _full(c3.shape),
        _full(w4.shape), _full(c4.shape), _full(w5.shape), _full(c5.shape),
    ]
    return pl.pallas_call(
        _mlp_body,
        grid=grid,
        in_specs=in_specs,
        out_specs=pl.BlockSpec((block_b, nlab), lambda i: (i, 0)),
        out_shape=jax.ShapeDtypeStruct((BATCH, nlab), jnp.float32),
        compiler_params=pltpu.CompilerParams(
            dimension_semantics=("arbitrary",)),
    )(inputs, g, a1, w1n, c1, w2, c2, w3, c3, w4, c4, w5, c5)


def kernel(inputs, speed_emb, speed_bias, oneway_emb, oneway_bias, lane_emb,
           lane_bias, W1, b1, W2, b2, W3, b3, W4, b4, W5, b5):
    idx3 = inputs[:, 0:3].astype(jnp.int32)
    idxw = (idx3.T.reshape(3, _NW, _IPR, 128)
            .transpose(1, 0, 2, 3).reshape(_NW, 3 * _IPR, 128))

    t0 = speed_emb.reshape(-1, 128)
    t1 = oneway_emb.reshape(-1, 128)
    t2 = lane_emb.reshape(-1, 128)

    def padbias(bias):
        v = bias.reshape(-1)
        n = v.shape[0]
        pad = (-n) % 128
        return jnp.pad(v, (0, pad)).reshape(-1, 128)

    b0 = padbias(speed_bias)
    b1v = padbias(oneway_bias)
    b2v = padbias(lane_bias)

    g = _sc_gather(idxw, t0, t1, t2, b0, b1v, b2v)

    w1s = jnp.sum(W1[:3 * EMB], axis=0, keepdims=True)
    a1 = jnp.concatenate([W1[0:EMB], w1s, W1[EMB:2 * EMB], w1s,
                          W1[2 * EMB:3 * EMB], w1s], axis=0)
    w1n = W1[3 * EMB:]

    return _mlp(inputs, g, a1, w1n,
                b1.reshape(1, -1), W2, b2.reshape(1, -1),
                W3, b3.reshape(1, -1), W4, b4.reshape(1, -1),
                W5, b5.reshape(1, -1), block_b=2048)


# speedup vs baseline: 4.3806x; 1.0151x over previous
import functools

import jax
import jax.numpy as jnp
from jax import lax
from jax.experimental import pallas as pl
from jax.experimental.pallas import tpu as pltpu
from jax.experimental.pallas import tpu_sc as plsc

BATCH = 16384
EMB = 16
NUM_NUM = 16
UNITS = 128
GCOLS = 3 * (EMB + 1)

_NC = 2
_NS = 16
_NW = _NC * _NS
_BPW = BATCH // _NW
_L = 16
_IPR = 4


def _sc_gather_body(idx_hbm, t0, t1, t2, b0, b1, b2, g_hbm,
                    idx_v, t0v, t1v, t2v, b0v, b1v, b2v, g_v):
    wid = lax.axis_index("s") * _NC + lax.axis_index("c")
    pltpu.sync_copy(idx_hbm.at[wid], idx_v)
    pltpu.sync_copy(t0, t0v)
    pltpu.sync_copy(t1, t1v)
    pltpu.sync_copy(t2, t2v)
    pltpu.sync_copy(b0, b0v)
    pltpu.sync_copy(b1, b1v)
    pltpu.sync_copy(b2, b2v)

    tables = ((t0v, b0v, 0), (t1v, b1v, EMB + 1), (t2v, b2v, 2 * (EMB + 1)))
    for kk in range(_IPR):
        for c in range(8):
            k = kk * 8 + c
            for t, (tv, bv, row0) in enumerate(tables):
                iv = idx_v[t * _IPR + kk, pl.ds(c * _L, _L)]
                lin = iv * EMB
                for j in range(EMB):
                    lj = lin + j
                    vals = plsc.load_gather(
                        tv, [lax.shift_right_logical(lj, 7),
                             lax.bitwise_and(lj, 127)])
                    g_v[row0 + j, pl.ds(k * _L, _L)] = vals
                bvals = plsc.load_gather(
                    bv, [lax.shift_right_logical(iv, 7),
                         lax.bitwise_and(iv, 127)])
                g_v[row0 + EMB, pl.ds(k * _L, _L)] = bvals

    pltpu.sync_copy(g_v, g_hbm.at[:, pl.ds(wid * _BPW, _BPW)])


def _sc_gather(idxw, t0, t1, t2, b0, b1, b2):
    mesh = plsc.VectorSubcoreMesh(core_axis_name="c", subcore_axis_name="s")
    f = functools.partial(
        pl.kernel,
        mesh=mesh,
        out_type=jax.ShapeDtypeStruct((GCOLS, BATCH), jnp.float32),
        scratch_types=[
            pltpu.VMEM((3 * _IPR, 128), jnp.int32),
            pltpu.VMEM(t0.shape, jnp.float32),
            pltpu.VMEM(t1.shape, jnp.float32),
            pltpu.VMEM(t2.shape, jnp.float32),
            pltpu.VMEM(b0.shape, jnp.float32),
            pltpu.VMEM(b1.shape, jnp.float32),
            pltpu.VMEM(b2.shape, jnp.float32),
            pltpu.VMEM((GCOLS, _BPW), jnp.float32),
        ],
        compiler_params=pltpu.CompilerParams(needs_layout_passes=False),
    )(_sc_gather_body)
    return f(idxw, t0, t1, t2, b0, b1, b2)


def _mlp_body(inp_ref, g_ref, a1, w1n, c1, w2, c2, w3, c3, w4, c4, w5, c5,
              out_ref):
    dot = functools.partial(jnp.dot, preferred_element_type=jnp.float32)
    num = inp_ref[:, 3:3 + NUM_NUM]
    h = lax.dot_general(g_ref[...], a1[...], (((0,), (0,)), ((), ())),
                        preferred_element_type=jnp.float32)
    h = h + dot(num, w1n[...]) + c1[...]
    h = jnp.maximum(h, 0.0)
    h = jnp.maximum(dot(h, w2[...]) + c2[...], 0.0)
    h = jnp.maximum(dot(h, w3[...]) + c3[...], 0.0)
    h = jnp.maximum(dot(h, w4[...]) + c4[...], 0.0)
    logits = dot(h, w5[...]) + c5[...]
    m = jnp.max(logits, axis=-1, keepdims=True)
    e = jnp.exp(logits - m)
    out_ref[...] = e / jnp.sum(e, axis=-1, keepdims=True)


def _full(shape):
    return pl.BlockSpec(shape, lambda i: (0, 0))


def _mlp(inputs, g, a1, w1n, c1, w2, c2, w3, c3, w4, c4, w5, c5, block_b):
    nlab = w5.shape[1]
    grid = (BATCH // block_b,)
    in_specs = [
        pl.BlockSpec((block_b, inputs.shape[1]), lambda i: (i, 0)),
        pl.BlockSpec((GCOLS, block_b), lambda i: (0, i)),
        _full(a1.shape), _full(w1n.shape), _full(c1.shape),
        _full(w2.shape), _full(c2.shape), _full(w3.shape), _full(c3.shape),
        _full(w4.shape), _full(c4.shape), _full(w5.shape), _full(c5.shape),
    ]
    return pl.pallas_call(
        _mlp_body,
        grid=grid,
        in_specs=in_specs,
        out_specs=pl.BlockSpec((block_b, nlab), lambda i: (i, 0)),
        out_shape=jax.ShapeDtypeStruct((BATCH, nlab), jnp.float32),
        compiler_params=pltpu.CompilerParams(
            dimension_semantics=("arbitrary",)),
    )(inputs, g, a1, w1n, c1, w2, c2, w3, c3, w4, c4, w5, c5)


def kernel(inputs, speed_emb, speed_bias, oneway_emb, oneway_bias, lane_emb,
           lane_bias, W1, b1, W2, b2, W3, b3, W4, b4, W5, b5):
    idx3 = inputs[:, 0:3].astype(jnp.int32)
    idxw = (idx3.T.reshape(3, _NW, _IPR, 128)
            .transpose(1, 0, 2, 3).reshape(_NW, 3 * _IPR, 128))

    t0 = speed_emb.reshape(-1, 128)
    t1 = oneway_emb.reshape(-1, 128)
    t2 = lane_emb.reshape(-1, 128)

    def padbias(bias):
        v = bias.reshape(-1)
        n = v.shape[0]
        pad = (-n) % 128
        return jnp.pad(v, (0, pad)).reshape(-1, 128)

    b0 = padbias(speed_bias)
    b1v = padbias(oneway_bias)
    b2v = padbias(lane_bias)

    g = _sc_gather(idxw, t0, t1, t2, b0, b1v, b2v)

    w1s = jnp.sum(W1[:3 * EMB], axis=0, keepdims=True)
    a1 = jnp.concatenate([W1[0:EMB], w1s, W1[EMB:2 * EMB], w1s,
                          W1[2 * EMB:3 * EMB], w1s], axis=0)
    w1n = W1[3 * EMB:]

    return _mlp(inputs, g, a1, w1n,
                b1.reshape(1, -1), W2, b2.reshape(1, -1),
                W3, b3.reshape(1, -1), W4, b4.reshape(1, -1),
                W5, b5.reshape(1, -1), block_b=4096)
